# Initial kernel scaffold; baseline (speedup 1.0000x reference)
import functools

import jax
import jax.numpy as jnp
from jax import lax
from jax.experimental import pallas as pl
from jax.experimental.pallas import tpu as pltpu
from jax.experimental.pallas import tpu_sc as plsc

N_NODES = 10000
N_EDGES = 320000
D = 128

NC = 2
NS = 16
NW = NC * NS
E_PER_TILE = N_EDGES // NW
CH = 80
NCHUNK = E_PER_TILE // CH
ROWS_PER_TILE = N_NODES // NS


def _sc_scatter_add():
    mesh = plsc.VectorSubcoreMesh(
        core_axis_name="c", subcore_axis_name="s", num_cores=NC, num_subcores=NS
    )

    @functools.partial(
        pl.kernel,
        out_type=jax.ShapeDtypeStruct((NC, N_NODES, D), jnp.float32),
        mesh=mesh,
        scratch_types=[
            pltpu.VMEM_SHARED((N_NODES, D), jnp.float32),
            pltpu.VMEM((NCHUNK, CH), jnp.int32),
            pltpu.VMEM((NCHUNK, CH), jnp.int32),
            pltpu.VMEM((CH, D), jnp.float32),
        ],
    )
    def sc_kernel(x_hbm, src_hbm, dst_hbm, zeros_hbm, out_hbm,
                  acc, srcv, dstv, rows):
        c = lax.axis_index("c")
        s = lax.axis_index("s")
        wid = c * NS + s

        pltpu.sync_copy(zeros_hbm, acc.at[pl.ds(s * ROWS_PER_TILE, ROWS_PER_TILE)])

        pltpu.sync_copy(src_hbm.at[wid], srcv)
        pltpu.sync_copy(dst_hbm.at[wid], dstv)

        plsc.subcore_barrier()

        def chunk(i, carry):
            pltpu.sync_copy(x_hbm.at[srcv.at[i]], rows)
            pltpu.sync_copy(rows, acc.at[dstv.at[i]], add=True)
            return carry

        lax.fori_loop(0, NCHUNK, chunk, 0)

        plsc.subcore_barrier()

        pltpu.sync_copy(
            acc.at[pl.ds(s * ROWS_PER_TILE, ROWS_PER_TILE)],
            out_hbm.at[c, pl.ds(s * ROWS_PER_TILE, ROWS_PER_TILE)],
        )

    return sc_kernel


def _tc_combine_matmul(partials, W, b):
    BLK = 1000

    def tc_body(p_ref, w_ref, b_ref, o_ref):
        acc = p_ref[0] + p_ref[1]
        o_ref[...] = (
            jnp.dot(acc, w_ref[...], preferred_element_type=jnp.float32)
            + b_ref[...]
        )

    return pl.pallas_call(
        tc_body,
        grid=(N_NODES // BLK,),
        in_specs=[
            pl.BlockSpec((NC, BLK, D), lambda i: (0, i, 0)),
            pl.BlockSpec((D, D), lambda i: (0, 0)),
            pl.BlockSpec((1, D), lambda i: (0, 0)),
        ],
        out_specs=pl.BlockSpec((BLK, D), lambda i: (i, 0)),
        out_shape=jax.ShapeDtypeStruct((N_NODES, D), jnp.float32),
    )(partials, W, b.reshape(1, D))


def kernel(x, edge_index, W, b):
    src = edge_index[0].astype(jnp.int32).reshape(NW, NCHUNK, CH)
    dst = edge_index[1].astype(jnp.int32).reshape(NW, NCHUNK, CH)
    zeros = jnp.zeros((ROWS_PER_TILE, D), jnp.float32)
    partials = _sc_scatter_add()(x, src, dst, zeros)
    return _tc_combine_matmul(partials, W, b)


# speedup vs baseline: 7.6170x; 7.6170x over previous
import functools

import jax
import jax.numpy as jnp
from jax import lax
from jax.experimental import pallas as pl
from jax.experimental.pallas import tpu as pltpu
from jax.experimental.pallas import tpu_sc as plsc

N_NODES = 10000
N_EDGES = 320000
D = 128

NC = 2
NS = 16
NW = NC * NS
E_PER_TILE = N_EDGES // NW
CH = 80
NCHUNK = E_PER_TILE // CH
STRIPE = 624
REM = N_NODES - NS * STRIPE


def _sc_scatter_add():
    mesh = plsc.VectorSubcoreMesh(
        core_axis_name="c", subcore_axis_name="s", num_cores=NC, num_subcores=NS
    )

    @functools.partial(
        pl.kernel,
        out_type=jax.ShapeDtypeStruct((NC, N_NODES, D), jnp.float32),
        mesh=mesh,
        scratch_types=[
            pltpu.VMEM_SHARED((N_NODES, D), jnp.float32),
            pltpu.VMEM((NCHUNK, CH), jnp.int32),
            pltpu.VMEM((NCHUNK, CH), jnp.int32),
            pltpu.VMEM((CH, D), jnp.float32),
        ],
    )
    def sc_kernel(x_hbm, src_hbm, dst_hbm, zeros_hbm, out_hbm,
                  acc, srcv, dstv, rows):
        c = lax.axis_index("c")
        s = lax.axis_index("s")
        wid = c * NS + s

        pltpu.sync_copy(zeros_hbm, acc.at[pl.ds(s * STRIPE, STRIPE)])

        @pl.when(s == NS - 1)
        def _():
            pltpu.sync_copy(
                zeros_hbm.at[pl.ds(0, REM)],
                acc.at[pl.ds(NS * STRIPE, REM)],
            )

        pltpu.sync_copy(src_hbm.at[wid], srcv)
        pltpu.sync_copy(dst_hbm.at[wid], dstv)

        plsc.subcore_barrier()

        def chunk(i, carry):
            pltpu.sync_copy(x_hbm.at[srcv.at[i]], rows)
            pltpu.sync_copy(rows, acc.at[dstv.at[i]], add=True)
            return carry

        lax.fori_loop(0, NCHUNK, chunk, 0)

        plsc.subcore_barrier()

        pltpu.sync_copy(
            acc.at[pl.ds(s * STRIPE, STRIPE)],
            out_hbm.at[c, pl.ds(s * STRIPE, STRIPE)],
        )

        @pl.when(s == NS - 1)
        def _():
            pltpu.sync_copy(
                acc.at[pl.ds(NS * STRIPE, REM)],
                out_hbm.at[c, pl.ds(NS * STRIPE, REM)],
            )

    return sc_kernel


def _tc_combine_matmul(partials, W, b):
    BLK = 1000

    def tc_body(p_ref, w_ref, b_ref, o_ref):
        acc = p_ref[0] + p_ref[1]
        o_ref[...] = (
            jnp.dot(acc, w_ref[...], preferred_element_type=jnp.float32)
            + b_ref[...]
        )

    return pl.pallas_call(
        tc_body,
        grid=(N_NODES // BLK,),
        in_specs=[
            pl.BlockSpec((NC, BLK, D), lambda i: (0, i, 0)),
            pl.BlockSpec((D, D), lambda i: (0, 0)),
            pl.BlockSpec((1, D), lambda i: (0, 0)),
        ],
        out_specs=pl.BlockSpec((BLK, D), lambda i: (i, 0)),
        out_shape=jax.ShapeDtypeStruct((N_NODES, D), jnp.float32),
    )(partials, W, b.reshape(1, D))


def kernel(x, edge_index, W, b):
    src = edge_index[0].astype(jnp.int32).reshape(NW, NCHUNK, CH)
    dst = edge_index[1].astype(jnp.int32).reshape(NW, NCHUNK, CH)
    zeros = jnp.zeros((STRIPE, D), jnp.float32)
    partials = _sc_scatter_add()(x, src, dst, zeros)
    return _tc_combine_matmul(partials, W, b)
